# baseline (speedup 1.0000x reference)
import functools

import jax
import jax.numpy as jnp
from jax import lax
from jax.experimental import pallas as pl
from jax.experimental.pallas import tpu as pltpu
from jax.experimental.pallas import tpu_sc as plsc

N = 10000
NP = 10240
E = 320000
EP = 327680
EPA = 331776
H = 128
F = 9
V = 119
G = 64
NB = 1024
GRID = NP // NB

_f32 = jnp.float32
_i32 = jnp.int32

_MESH = plsc.VectorSubcoreMesh(core_axis_name="c", subcore_axis_name="s",
                               num_cores=2, num_subcores=16)


def _sc1_body(xt_hbm, embf_hbm, h0_out,
              xbuf, idxall, embrows9, outbuf, sem):
    cid = lax.axis_index("c")
    sid = lax.axis_index("s")
    wid = cid * 16 + sid
    nb = wid * 320

    def chunk(cc, _):
        base = nb + cc * 32
        for i in range(F):
            pltpu.sync_copy(xt_hbm.at[pl.ds(i * NP + base, 32)], xbuf)
            for k in range(2):
                idxall[pl.ds(i * 32 + k * 16, 16)] = (
                    xbuf[pl.ds(k * 16, 16)] + i * V)
        for i in range(F):
            pltpu.async_copy(embf_hbm.at[idxall.at[pl.ds(i * 32, 32)]],
                             embrows9.at[pl.ds(i * 32, 32)], sem)
        for i in range(F):
            pltpu.make_async_copy(embf_hbm.at[pl.ds(0, 32)],
                                  embrows9.at[pl.ds(i * 32, 32)], sem).wait()

        def row(r, _):
            for k in range(8):
                s = pl.ds(k * 16, 16)
                v = embrows9[r, s]
                for i in range(1, F):
                    v = v + embrows9[i * 32 + r, s]
                outbuf[r, s] = v
            return 0
        lax.fori_loop(0, 32, row, 0)
        pltpu.sync_copy(outbuf, h0_out.at[pl.ds(base, 32)])
        return 0
    lax.fori_loop(0, 10, chunk, 0)


_sc1 = functools.partial(
    pl.kernel,
    out_type=jax.ShapeDtypeStruct((NP, H), _f32),
    mesh=_MESH,
    scratch_types=[
        pltpu.VMEM((32,), _i32),
        pltpu.VMEM((288,), _i32),
        pltpu.VMEM((288, 128), _f32),
        pltpu.VMEM((32, 128), _f32),
        pltpu.SemaphoreType.DMA,
    ],
)(_sc1_body)


_SLOW_CORE = 0
_RS = 24
_RF = 136

def _agg_body(xwp_hbm, src2d_hbm, dst2d_hbm, agg_out,
              sidxb, didxb, rowa, rowb, acc_sh, sema, semb):
    cid = lax.axis_index("c")
    sid = lax.axis_index("s")

    def zrow(t, _):
        for k in range(8):
            rowa[t, pl.ds(k * 16, 16)] = jnp.zeros((16,), _f32)
        return 0
    lax.fori_loop(0, 128, zrow, 0)
    for m in range(5):
        pltpu.sync_copy(rowa, acc_sh.at[pl.ds(sid * 640 + m * 128, 128)])
    plsc.subcore_barrier()

    rmine = jnp.where(cid == _SLOW_CORE, _RS, _RF)
    rowbase = jnp.where(cid == _SLOW_CORE, sid * _RS, 16 * _RS + sid * _RF)
    for q in range(4):

        @pl.when(q * 40 < rmine)
        def _():
            cnt = jnp.minimum(40, rmine - q * 40)
            pltpu.sync_copy(src2d_hbm.at[pl.ds(rowbase + q * 40, 40)], sidxb)
            pltpu.sync_copy(dst2d_hbm.at[pl.ds(rowbase + q * 40, 40)], didxb)
            pltpu.async_copy(xwp_hbm.at[sidxb.at[0]], rowa, sema)

            def step(j, _):
                even = lax.rem(j, 2) == 0

                @pl.when(even)
                def _():
                    @pl.when(j + 1 < cnt)
                    def _():
                        pltpu.async_copy(xwp_hbm.at[sidxb.at[j + 1]],
                                         rowb, semb)
                    pltpu.make_async_copy(xwp_hbm.at[pl.ds(0, 128)],
                                          rowa, sema).wait()
                    pltpu.sync_copy(rowa, acc_sh.at[didxb.at[j]], add=True)

                @pl.when(jnp.logical_not(even))
                def _():
                    @pl.when(j + 1 < cnt)
                    def _():
                        pltpu.async_copy(xwp_hbm.at[sidxb.at[j + 1]],
                                         rowa, sema)
                    pltpu.make_async_copy(xwp_hbm.at[pl.ds(0, 128)],
                                          rowb, semb).wait()
                    pltpu.sync_copy(rowb, acc_sh.at[didxb.at[j]], add=True)
                return 0
            lax.fori_loop(0, cnt, step, 0)

    plsc.subcore_barrier()
    for m in range(5):
        pltpu.sync_copy(acc_sh.at[pl.ds(sid * 640 + m * 128, 128)],
                        agg_out.at[pl.ds(cid * NP + sid * 640 + m * 128, 128)])


_sc_agg = functools.partial(
    pl.kernel,
    out_type=jax.ShapeDtypeStruct((2 * NP, H), _f32),
    mesh=_MESH,
    scratch_types=[
        pltpu.VMEM((40, 128), _i32),
        pltpu.VMEM((40, 128), _i32),
        pltpu.VMEM((128, 128), _f32),
        pltpu.VMEM((128, 128), _f32),
        pltpu.VMEM_SHARED((NP, H), _f32),
        pltpu.SemaphoreType.DMA,
        pltpu.SemaphoreType.DMA,
    ],
)(_agg_body)


def _mm_body(h_ref, w_ref, dinv_ref, o_ref):
    o_ref[...] = jnp.dot(h_ref[...], w_ref[...],
                         preferred_element_type=_f32) * dinv_ref[...]


def _tc_matmul(h, w, dinv):
    return pl.pallas_call(
        _mm_body,
        grid=(GRID,),
        in_specs=[pl.BlockSpec((NB, H), lambda i: (i, 0)),
                  pl.BlockSpec((H, H), lambda i: (0, 0)),
                  pl.BlockSpec((NB, 1), lambda i: (i, 0))],
        out_specs=pl.BlockSpec((NB, H), lambda i: (i, 0)),
        out_shape=jax.ShapeDtypeStruct((NP, H), _f32),
    )(h, w, dinv)


def _stats_body(a0_ref, a1_ref, xwp_ref, dinv_ref, b_ref, t_ref, s_ref):
    i = pl.program_id(0)
    tt = (a0_ref[...] + a1_ref[...] + xwp_ref[...]) * dinv_ref[...] + b_ref[...]
    rows = i * NB + lax.broadcasted_iota(_i32, (NB, 1), 0)
    tt = jnp.where(rows < N, tt, 0.0)
    t_ref[...] = tt

    @pl.when(i == 0)
    def _():
        s_ref[...] = jnp.zeros_like(s_ref)
    s_ref[...] += jnp.stack([tt.sum(0), (tt * tt).sum(0)])


def _tc_stats(agg, xwp, dinv, bias):
    return pl.pallas_call(
        _stats_body,
        grid=(GRID,),
        in_specs=[pl.BlockSpec((NB, H), lambda i: (i, 0)),
                  pl.BlockSpec((NB, H), lambda i: (i + GRID, 0)),
                  pl.BlockSpec((NB, H), lambda i: (i, 0)),
                  pl.BlockSpec((NB, 1), lambda i: (i, 0)),
                  pl.BlockSpec((1, H), lambda i: (0, 0))],
        out_specs=[pl.BlockSpec((NB, H), lambda i: (i, 0)),
                   pl.BlockSpec((2, H), lambda i: (0, 0))],
        out_shape=[jax.ShapeDtypeStruct((NP, H), _f32),
                   jax.ShapeDtypeStruct((2, H), _f32)],
    )(agg, agg, xwp, dinv, bias)


def _next_body(t_ref, res_ref, a_ref, c_ref, w_ref, dinv_ref, h_ref, xw_ref):
    h = jnp.maximum(t_ref[...] * a_ref[...] + c_ref[...], 0.0) + res_ref[...]
    h_ref[...] = h
    xw_ref[...] = jnp.dot(h, w_ref[...],
                          preferred_element_type=_f32) * dinv_ref[...]


def _tc_next(t, res, a, c, w, dinv):
    return pl.pallas_call(
        _next_body,
        grid=(GRID,),
        in_specs=[pl.BlockSpec((NB, H), lambda i: (i, 0)),
                  pl.BlockSpec((NB, H), lambda i: (i, 0)),
                  pl.BlockSpec((1, H), lambda i: (0, 0)),
                  pl.BlockSpec((1, H), lambda i: (0, 0)),
                  pl.BlockSpec((H, H), lambda i: (0, 0)),
                  pl.BlockSpec((NB, 1), lambda i: (i, 0))],
        out_specs=[pl.BlockSpec((NB, H), lambda i: (i, 0)),
                   pl.BlockSpec((NB, H), lambda i: (i, 0))],
        out_shape=[jax.ShapeDtypeStruct((NP, H), _f32),
                   jax.ShapeDtypeStruct((NP, H), _f32)],
    )(t, res, a, c, w, dinv)


def _ro_body(t_ref, res_ref, a_ref, c_ref, bi_ref, lw_ref, lb_ref,
             o_ref, acc_s, acc_c):
    i = pl.program_id(0)
    h = t_ref[...] * a_ref[...] + c_ref[...] + res_ref[...]
    rows = i * NB + lax.broadcasted_iota(_i32, (NB, 1), 0)
    m = rows < N
    oh = jnp.where((bi_ref[...] == lax.broadcasted_iota(_i32, (NB, G), 1)) & m,
                   1.0, 0.0)

    @pl.when(i == 0)
    def _():
        acc_s[...] = jnp.zeros_like(acc_s)
        acc_c[...] = jnp.zeros_like(acc_c)

    dn = (((0,), (0,)), ((), ()))
    acc_s[...] += lax.dot_general(oh, h, dn, preferred_element_type=_f32)
    acc_c[...] += lax.dot_general(oh, jnp.ones((NB, 1), _f32), dn,
                                  preferred_element_type=_f32)

    @pl.when(i == GRID - 1)
    def _():
        g = acc_s[...] / jnp.maximum(acc_c[...], 1.0)
        o_ref[...] = jax.nn.sigmoid(
            jnp.dot(g, lw_ref[...], preferred_element_type=_f32) + lb_ref[...])


def _tc_readout(t, res, a, c, bi, lw, lb):
    return pl.pallas_call(
        _ro_body,
        grid=(GRID,),
        in_specs=[pl.BlockSpec((NB, H), lambda i: (i, 0)),
                  pl.BlockSpec((NB, H), lambda i: (i, 0)),
                  pl.BlockSpec((1, H), lambda i: (0, 0)),
                  pl.BlockSpec((1, H), lambda i: (0, 0)),
                  pl.BlockSpec((NB, 1), lambda i: (i, 0)),
                  pl.BlockSpec((H, 1), lambda i: (0, 0)),
                  pl.BlockSpec((1, 1), lambda i: (0, 0))],
        out_specs=pl.BlockSpec((G, 1), lambda i: (0, 0)),
        out_shape=jax.ShapeDtypeStruct((G, 1), _f32),
        scratch_shapes=[pltpu.VMEM((G, H), _f32), pltpu.VMEM((G, 1), _f32)],
    )(t, res, a, c, bi, lw, lb)


def kernel(x, edge_index, batch_idx, emb, W, b, gamma, beta, lin_W, lin_b):
    xt = jnp.pad(x.astype(_i32).T, ((0, 0), (0, NP - N))).reshape(-1)
    embf = emb.reshape(F * V, H)
    src2d = jnp.pad(edge_index[0].astype(_i32), (0, EPA - E),
                    constant_values=N).reshape(EPA // 128, 128)
    dst2d = jnp.pad(edge_index[1].astype(_i32), (0, EPA - E),
                    constant_values=N).reshape(EPA // 128, 128)
    bi = jnp.pad(batch_idx.astype(_i32), (0, NP - N)).reshape(NP, 1)

    h0 = _sc1(xt, embf)
    deg2 = _sc_agg(jnp.ones((NP, H), _f32), src2d, dst2d)
    deg = deg2[:NP, 0] + deg2[NP:, 0] + 1.0
    dinv = lax.rsqrt(deg).reshape(NP, 1)

    xwp = _tc_matmul(h0, W[0], dinv)
    hprev = h0
    for l in range(3):
        agg = _sc_agg(xwp, src2d, dst2d)
        t, sums = _tc_stats(agg, xwp, dinv, b[l].reshape(1, H))
        mu = sums[0] / N
        var = sums[1] / N - mu * mu
        aff_a = (gamma[l] * lax.rsqrt(var + 1e-5)).reshape(1, H)
        aff_c = (beta[l] - mu * gamma[l] * lax.rsqrt(var + 1e-5)).reshape(1, H)
        if l < 2:
            hprev, xwp = _tc_next(t, hprev, aff_a, aff_c, W[l + 1], dinv)
        else:
            out = _tc_readout(t, hprev, aff_a, aff_c, bi,
                              lin_W, lin_b.reshape(1, 1))
    return out


# speedup vs baseline: 1.5620x; 1.5620x over previous
import functools

import jax
import jax.numpy as jnp
from jax import lax
from jax.experimental import pallas as pl
from jax.experimental.pallas import tpu as pltpu
from jax.experimental.pallas import tpu_sc as plsc

N = 10000
NP = 10240
E = 320000
EP = 327680
EPA = 331776
H = 128
F = 9
V = 119
G = 64
NB = 1024
GRID = NP // NB

_f32 = jnp.float32
_i32 = jnp.int32

_MESH = plsc.VectorSubcoreMesh(core_axis_name="c", subcore_axis_name="s",
                               num_cores=2, num_subcores=16)


def _sc1_body(xt_hbm, embf_hbm, h0_out,
              xbuf, idxall, embrows9, outbuf, sem):
    cid = lax.axis_index("c")
    sid = lax.axis_index("s")
    wid = cid * 16 + sid
    nb = wid * 320

    def chunk(cc, _):
        base = nb + cc * 32
        for i in range(F):
            pltpu.sync_copy(xt_hbm.at[pl.ds(i * NP + base, 32)], xbuf)
            for k in range(2):
                idxall[pl.ds(i * 32 + k * 16, 16)] = (
                    xbuf[pl.ds(k * 16, 16)] + i * V)
        for i in range(F):
            pltpu.async_copy(embf_hbm.at[idxall.at[pl.ds(i * 32, 32)]],
                             embrows9.at[pl.ds(i * 32, 32)], sem)
        for i in range(F):
            pltpu.make_async_copy(embf_hbm.at[pl.ds(0, 32)],
                                  embrows9.at[pl.ds(i * 32, 32)], sem).wait()

        def row(r, _):
            for k in range(8):
                s = pl.ds(k * 16, 16)
                v = embrows9[r, s]
                for i in range(1, F):
                    v = v + embrows9[i * 32 + r, s]
                outbuf[r, s] = v
            return 0
        lax.fori_loop(0, 32, row, 0)
        pltpu.sync_copy(outbuf, h0_out.at[pl.ds(base, 32)])
        return 0
    lax.fori_loop(0, 10, chunk, 0)


_sc1 = functools.partial(
    pl.kernel,
    out_type=jax.ShapeDtypeStruct((NP, H), _f32),
    mesh=_MESH,
    scratch_types=[
        pltpu.VMEM((32,), _i32),
        pltpu.VMEM((288,), _i32),
        pltpu.VMEM((288, 128), _f32),
        pltpu.VMEM((32, 128), _f32),
        pltpu.SemaphoreType.DMA,
    ],
)(_sc1_body)


def _agg_body(xwp_hbm, src2d_hbm, dst2d_hbm, agg_out,
              sidxb, didxb, rowa, rowb, acc_sh, sema, semb):
    cid = lax.axis_index("c")
    sid = lax.axis_index("s")

    @pl.when(cid == 0)
    def _():
        def zrow(t, _):
            for k in range(8):
                rowa[t, pl.ds(k * 16, 16)] = jnp.zeros((16,), _f32)
            return 0
        lax.fori_loop(0, 128, zrow, 0)
        for m in range(5):
            pltpu.sync_copy(rowa, acc_sh.at[pl.ds(sid * 640 + m * 128, 128)])
        plsc.subcore_barrier()

        for q in range(4):
            rowbase = sid * 160 + q * 40
            pltpu.sync_copy(src2d_hbm.at[pl.ds(rowbase, 40)], sidxb)
            pltpu.sync_copy(dst2d_hbm.at[pl.ds(rowbase, 40)], didxb)
            pltpu.async_copy(xwp_hbm.at[sidxb.at[0]], rowa, sema)

            def step(j, _):
                even = lax.rem(j, 2) == 0

                @pl.when(even)
                def _():
                    @pl.when(j < 39)
                    def _():
                        pltpu.async_copy(xwp_hbm.at[sidxb.at[j + 1]],
                                         rowb, semb)
                    pltpu.make_async_copy(xwp_hbm.at[pl.ds(0, 128)],
                                          rowa, sema).wait()
                    pltpu.sync_copy(rowa, acc_sh.at[didxb.at[j]], add=True)

                @pl.when(jnp.logical_not(even))
                def _():
                    @pl.when(j < 39)
                    def _():
                        pltpu.async_copy(xwp_hbm.at[sidxb.at[j + 1]],
                                         rowa, sema)
                    pltpu.make_async_copy(xwp_hbm.at[pl.ds(0, 128)],
                                          rowb, semb).wait()
                    pltpu.sync_copy(rowb, acc_sh.at[didxb.at[j]], add=True)
                return 0
            lax.fori_loop(0, 40, step, 0)

        plsc.subcore_barrier()
        for m in range(5):
            pltpu.sync_copy(acc_sh.at[pl.ds(sid * 640 + m * 128, 128)],
                            agg_out.at[pl.ds(sid * 640 + m * 128, 128)])


_sc_agg = functools.partial(
    pl.kernel,
    out_type=jax.ShapeDtypeStruct((NP, H), _f32),
    mesh=_MESH,
    scratch_types=[
        pltpu.VMEM((40, 128), _i32),
        pltpu.VMEM((40, 128), _i32),
        pltpu.VMEM((128, 128), _f32),
        pltpu.VMEM((128, 128), _f32),
        pltpu.VMEM_SHARED((NP, H), _f32),
        pltpu.SemaphoreType.DMA,
        pltpu.SemaphoreType.DMA,
    ],
)(_agg_body)


EB = 4096


def _deg_body(dst_ref, o_ref):
    i = pl.program_id(0)
    d = dst_ref[...]
    roh = jnp.where(d // 128 == lax.broadcasted_iota(_i32, (EB, 80), 1),
                    1.0, 0.0).astype(jnp.bfloat16)
    coh = jnp.where(d % 128 == lax.broadcasted_iota(_i32, (EB, H), 1),
                    1.0, 0.0).astype(jnp.bfloat16)

    @pl.when(i == 0)
    def _():
        o_ref[...] = jnp.zeros_like(o_ref)
    dn = (((0,), (0,)), ((), ()))
    o_ref[...] += lax.dot_general(roh, coh, dn, preferred_element_type=_f32)


def _tc_deg(dstf):
    return pl.pallas_call(
        _deg_body,
        grid=(EP // EB,),
        in_specs=[pl.BlockSpec((EB, 1), lambda i: (i, 0))],
        out_specs=pl.BlockSpec((80, H), lambda i: (0, 0)),
        out_shape=jax.ShapeDtypeStruct((80, H), _f32),
    )(dstf)


def _mm_body(h_ref, w_ref, dinv_ref, o_ref):
    o_ref[...] = jnp.dot(h_ref[...], w_ref[...],
                         preferred_element_type=_f32) * dinv_ref[...]


def _tc_matmul(h, w, dinv):
    return pl.pallas_call(
        _mm_body,
        grid=(GRID,),
        in_specs=[pl.BlockSpec((NB, H), lambda i: (i, 0)),
                  pl.BlockSpec((H, H), lambda i: (0, 0)),
                  pl.BlockSpec((NB, 1), lambda i: (i, 0))],
        out_specs=pl.BlockSpec((NB, H), lambda i: (i, 0)),
        out_shape=jax.ShapeDtypeStruct((NP, H), _f32),
    )(h, w, dinv)


def _stats_body(a0_ref, xwp_ref, dinv_ref, b_ref, t_ref, s_ref):
    i = pl.program_id(0)
    tt = (a0_ref[...] + xwp_ref[...]) * dinv_ref[...] + b_ref[...]
    rows = i * NB + lax.broadcasted_iota(_i32, (NB, 1), 0)
    tt = jnp.where(rows < N, tt, 0.0)
    t_ref[...] = tt

    @pl.when(i == 0)
    def _():
        s_ref[...] = jnp.zeros_like(s_ref)
    s_ref[...] += jnp.stack([tt.sum(0), (tt * tt).sum(0)])


def _tc_stats(agg, xwp, dinv, bias):
    return pl.pallas_call(
        _stats_body,
        grid=(GRID,),
        in_specs=[pl.BlockSpec((NB, H), lambda i: (i, 0)),
                  pl.BlockSpec((NB, H), lambda i: (i, 0)),
                  pl.BlockSpec((NB, 1), lambda i: (i, 0)),
                  pl.BlockSpec((1, H), lambda i: (0, 0))],
        out_specs=[pl.BlockSpec((NB, H), lambda i: (i, 0)),
                   pl.BlockSpec((2, H), lambda i: (0, 0))],
        out_shape=[jax.ShapeDtypeStruct((NP, H), _f32),
                   jax.ShapeDtypeStruct((2, H), _f32)],
    )(agg, xwp, dinv, bias)


def _next_body(t_ref, res_ref, a_ref, c_ref, w_ref, dinv_ref, h_ref, xw_ref):
    h = jnp.maximum(t_ref[...] * a_ref[...] + c_ref[...], 0.0) + res_ref[...]
    h_ref[...] = h
    xw_ref[...] = jnp.dot(h, w_ref[...],
                          preferred_element_type=_f32) * dinv_ref[...]


def _tc_next(t, res, a, c, w, dinv):
    return pl.pallas_call(
        _next_body,
        grid=(GRID,),
        in_specs=[pl.BlockSpec((NB, H), lambda i: (i, 0)),
                  pl.BlockSpec((NB, H), lambda i: (i, 0)),
                  pl.BlockSpec((1, H), lambda i: (0, 0)),
                  pl.BlockSpec((1, H), lambda i: (0, 0)),
                  pl.BlockSpec((H, H), lambda i: (0, 0)),
                  pl.BlockSpec((NB, 1), lambda i: (i, 0))],
        out_specs=[pl.BlockSpec((NB, H), lambda i: (i, 0)),
                   pl.BlockSpec((NB, H), lambda i: (i, 0))],
        out_shape=[jax.ShapeDtypeStruct((NP, H), _f32),
                   jax.ShapeDtypeStruct((NP, H), _f32)],
    )(t, res, a, c, w, dinv)


def _ro_body(t_ref, res_ref, a_ref, c_ref, bi_ref, lw_ref, lb_ref,
             o_ref, acc_s, acc_c):
    i = pl.program_id(0)
    h = t_ref[...] * a_ref[...] + c_ref[...] + res_ref[...]
    rows = i * NB + lax.broadcasted_iota(_i32, (NB, 1), 0)
    m = rows < N
    oh = jnp.where((bi_ref[...] == lax.broadcasted_iota(_i32, (NB, G), 1)) & m,
                   1.0, 0.0)

    @pl.when(i == 0)
    def _():
        acc_s[...] = jnp.zeros_like(acc_s)
        acc_c[...] = jnp.zeros_like(acc_c)

    dn = (((0,), (0,)), ((), ()))
    acc_s[...] += lax.dot_general(oh, h, dn, preferred_element_type=_f32)
    acc_c[...] += lax.dot_general(oh, jnp.ones((NB, 1), _f32), dn,
                                  preferred_element_type=_f32)

    @pl.when(i == GRID - 1)
    def _():
        g = acc_s[...] / jnp.maximum(acc_c[...], 1.0)
        o_ref[...] = jax.nn.sigmoid(
            jnp.dot(g, lw_ref[...], preferred_element_type=_f32) + lb_ref[...])


def _tc_readout(t, res, a, c, bi, lw, lb):
    return pl.pallas_call(
        _ro_body,
        grid=(GRID,),
        in_specs=[pl.BlockSpec((NB, H), lambda i: (i, 0)),
                  pl.BlockSpec((NB, H), lambda i: (i, 0)),
                  pl.BlockSpec((1, H), lambda i: (0, 0)),
                  pl.BlockSpec((1, H), lambda i: (0, 0)),
                  pl.BlockSpec((NB, 1), lambda i: (i, 0)),
                  pl.BlockSpec((H, 1), lambda i: (0, 0)),
                  pl.BlockSpec((1, 1), lambda i: (0, 0))],
        out_specs=pl.BlockSpec((G, 1), lambda i: (0, 0)),
        out_shape=jax.ShapeDtypeStruct((G, 1), _f32),
        scratch_shapes=[pltpu.VMEM((G, H), _f32), pltpu.VMEM((G, 1), _f32)],
    )(t, res, a, c, bi, lw, lb)


def kernel(x, edge_index, batch_idx, emb, W, b, gamma, beta, lin_W, lin_b):
    xt = jnp.pad(x.astype(_i32).T, ((0, 0), (0, NP - N))).reshape(-1)
    embf = emb.reshape(F * V, H)
    src2d = jnp.pad(edge_index[0].astype(_i32), (0, EPA - E),
                    constant_values=N).reshape(EPA // 128, 128)
    dst2d = jnp.pad(edge_index[1].astype(_i32), (0, EPA - E),
                    constant_values=N).reshape(EPA // 128, 128)
    bi = jnp.pad(batch_idx.astype(_i32), (0, NP - N)).reshape(NP, 1)

    dstf = jnp.pad(edge_index[1].astype(_i32), (0, EP - E),
                   constant_values=N).reshape(EP, 1)
    h0 = _sc1(xt, embf)
    deg = _tc_deg(dstf).reshape(NP) + 1.0
    dinv = lax.rsqrt(deg).reshape(NP, 1)

    xwp = _tc_matmul(h0, W[0], dinv)
    hprev = h0
    for l in range(3):
        agg = _sc_agg(xwp, src2d, dst2d)
        t, sums = _tc_stats(agg, xwp, dinv, b[l].reshape(1, H))
        mu = sums[0] / N
        var = sums[1] / N - mu * mu
        aff_a = (gamma[l] * lax.rsqrt(var + 1e-5)).reshape(1, H)
        aff_c = (beta[l] - mu * gamma[l] * lax.rsqrt(var + 1e-5)).reshape(1, H)
        if l < 2:
            hprev, xwp = _tc_next(t, hprev, aff_a, aff_c, W[l + 1], dinv)
        else:
            out = _tc_readout(t, hprev, aff_a, aff_c, bi,
                              lin_W, lin_b.reshape(1, 1))
    return out


# speedup vs baseline: 1.5632x; 1.0007x over previous
import functools

import jax
import jax.numpy as jnp
from jax import lax
from jax.experimental import pallas as pl
from jax.experimental.pallas import tpu as pltpu
from jax.experimental.pallas import tpu_sc as plsc

N = 10000
NP = 10240
E = 320000
EP = 327680
EPA = 331776
H = 128
F = 9
V = 119
G = 64
NB = 1024
GRID = NP // NB

_f32 = jnp.float32
_i32 = jnp.int32

_MESH = plsc.VectorSubcoreMesh(core_axis_name="c", subcore_axis_name="s",
                               num_cores=2, num_subcores=16)


def _sc1_body(xt_hbm, embf_hbm, h0_out,
              xbuf, idxall, embrows9, outbuf, sem):
    cid = lax.axis_index("c")
    sid = lax.axis_index("s")
    wid = cid * 16 + sid
    nb = wid * 320

    def chunk(cc, _):
        base = nb + cc * 32
        for i in range(F):
            pltpu.sync_copy(xt_hbm.at[pl.ds(i * NP + base, 32)], xbuf)
            for k in range(2):
                idxall[pl.ds(i * 32 + k * 16, 16)] = (
                    xbuf[pl.ds(k * 16, 16)] + i * V)
        for i in range(F):
            pltpu.async_copy(embf_hbm.at[idxall.at[pl.ds(i * 32, 32)]],
                             embrows9.at[pl.ds(i * 32, 32)], sem)
        for i in range(F):
            pltpu.make_async_copy(embf_hbm.at[pl.ds(0, 32)],
                                  embrows9.at[pl.ds(i * 32, 32)], sem).wait()

        def row(r, _):
            for k in range(8):
                s = pl.ds(k * 16, 16)
                v = embrows9[r, s]
                for i in range(1, F):
                    v = v + embrows9[i * 32 + r, s]
                outbuf[r, s] = v
            return 0
        lax.fori_loop(0, 32, row, 0)
        pltpu.sync_copy(outbuf, h0_out.at[pl.ds(base, 32)])
        return 0
    lax.fori_loop(0, 10, chunk, 0)


_sc1 = functools.partial(
    pl.kernel,
    out_type=jax.ShapeDtypeStruct((NP, H), _f32),
    mesh=_MESH,
    scratch_types=[
        pltpu.VMEM((32,), _i32),
        pltpu.VMEM((288,), _i32),
        pltpu.VMEM((288, 128), _f32),
        pltpu.VMEM((32, 128), _f32),
        pltpu.SemaphoreType.DMA,
    ],
)(_sc1_body)


def _agg_body(xwp_hbm, src2d_hbm, dst2d_hbm, agg_out,
              sidxb, didxb, rowa, rowb, acc_sh, sema, semb, semsa, semsb):
    cid = lax.axis_index("c")
    sid = lax.axis_index("s")

    @pl.when(cid == 0)
    def _():
        def zrow(t, _):
            for k in range(8):
                rowa[t, pl.ds(k * 16, 16)] = jnp.zeros((16,), _f32)
            return 0
        lax.fori_loop(0, 128, zrow, 0)
        for m in range(5):
            pltpu.sync_copy(rowa, acc_sh.at[pl.ds(sid * 640 + m * 128, 128)])
        plsc.subcore_barrier()

        for q in range(4):
            rowbase = sid * 160 + q * 40
            pltpu.sync_copy(src2d_hbm.at[pl.ds(rowbase, 40)], sidxb)
            pltpu.sync_copy(dst2d_hbm.at[pl.ds(rowbase, 40)], didxb)
            pltpu.async_copy(xwp_hbm.at[sidxb.at[0]], rowa, sema)

            def step(j, _):
                even = lax.rem(j, 2) == 0

                @pl.when(even)
                def _():
                    @pl.when(j > 0)
                    def _():
                        pltpu.make_async_copy(xwp_hbm.at[pl.ds(0, 128)],
                                              acc_sh.at[pl.ds(0, 128)],
                                              semsb).wait()

                    @pl.when(j < 39)
                    def _():
                        pltpu.async_copy(xwp_hbm.at[sidxb.at[j + 1]],
                                         rowb, semb)
                    pltpu.make_async_copy(xwp_hbm.at[pl.ds(0, 128)],
                                          rowa, sema).wait()
                    pltpu.async_copy(rowa, acc_sh.at[didxb.at[j]], semsa,
                                     add=True)

                @pl.when(jnp.logical_not(even))
                def _():
                    pltpu.make_async_copy(xwp_hbm.at[pl.ds(0, 128)],
                                          acc_sh.at[pl.ds(0, 128)],
                                          semsa).wait()

                    @pl.when(j < 39)
                    def _():
                        pltpu.async_copy(xwp_hbm.at[sidxb.at[j + 1]],
                                         rowa, sema)
                    pltpu.make_async_copy(xwp_hbm.at[pl.ds(0, 128)],
                                          rowb, semb).wait()
                    pltpu.async_copy(rowb, acc_sh.at[didxb.at[j]], semsb,
                                     add=True)
                return 0
            lax.fori_loop(0, 40, step, 0)
            pltpu.make_async_copy(xwp_hbm.at[pl.ds(0, 128)],
                                  acc_sh.at[pl.ds(0, 128)], semsb).wait()

        plsc.subcore_barrier()
        for m in range(5):
            pltpu.sync_copy(acc_sh.at[pl.ds(sid * 640 + m * 128, 128)],
                            agg_out.at[pl.ds(sid * 640 + m * 128, 128)])


_sc_agg = functools.partial(
    pl.kernel,
    out_type=jax.ShapeDtypeStruct((NP, H), _f32),
    mesh=_MESH,
    scratch_types=[
        pltpu.VMEM((40, 128), _i32),
        pltpu.VMEM((40, 128), _i32),
        pltpu.VMEM((128, 128), _f32),
        pltpu.VMEM((128, 128), _f32),
        pltpu.VMEM_SHARED((NP, H), _f32),
        pltpu.SemaphoreType.DMA,
        pltpu.SemaphoreType.DMA,
        pltpu.SemaphoreType.DMA,
        pltpu.SemaphoreType.DMA,
    ],
)(_agg_body)


EB = 8192


def _deg_body(dst_ref, o_ref):
    i = pl.program_id(0)
    d = dst_ref[...]
    roh = jnp.where(d // 128 == lax.broadcasted_iota(_i32, (EB, 80), 1),
                    1.0, 0.0).astype(jnp.bfloat16)
    coh = jnp.where(d % 128 == lax.broadcasted_iota(_i32, (EB, H), 1),
                    1.0, 0.0).astype(jnp.bfloat16)

    @pl.when(i == 0)
    def _():
        o_ref[...] = jnp.zeros_like(o_ref)
    dn = (((0,), (0,)), ((), ()))
    o_ref[...] += lax.dot_general(roh, coh, dn, preferred_element_type=_f32)


def _tc_deg(dstf):
    return pl.pallas_call(
        _deg_body,
        grid=(EP // EB,),
        in_specs=[pl.BlockSpec((EB, 1), lambda i: (i, 0))],
        out_specs=pl.BlockSpec((80, H), lambda i: (0, 0)),
        out_shape=jax.ShapeDtypeStruct((80, H), _f32),
    )(dstf)


def _mm_body(h_ref, w_ref, dinv_ref, o_ref):
    o_ref[...] = jnp.dot(h_ref[...], w_ref[...],
                         preferred_element_type=_f32) * dinv_ref[...]


def _tc_matmul(h, w, dinv):
    return pl.pallas_call(
        _mm_body,
        grid=(GRID,),
        in_specs=[pl.BlockSpec((NB, H), lambda i: (i, 0)),
                  pl.BlockSpec((H, H), lambda i: (0, 0)),
                  pl.BlockSpec((NB, 1), lambda i: (i, 0))],
        out_specs=pl.BlockSpec((NB, H), lambda i: (i, 0)),
        out_shape=jax.ShapeDtypeStruct((NP, H), _f32),
    )(h, w, dinv)


def _stats_body(a0_ref, xwp_ref, dinv_ref, b_ref, t_ref, s_ref):
    i = pl.program_id(0)
    tt = (a0_ref[...] + xwp_ref[...]) * dinv_ref[...] + b_ref[...]
    rows = i * NB + lax.broadcasted_iota(_i32, (NB, 1), 0)
    tt = jnp.where(rows < N, tt, 0.0)
    t_ref[...] = tt

    @pl.when(i == 0)
    def _():
        s_ref[...] = jnp.zeros_like(s_ref)
    s_ref[...] += jnp.stack([tt.sum(0), (tt * tt).sum(0)])


def _tc_stats(agg, xwp, dinv, bias):
    return pl.pallas_call(
        _stats_body,
        grid=(GRID,),
        in_specs=[pl.BlockSpec((NB, H), lambda i: (i, 0)),
                  pl.BlockSpec((NB, H), lambda i: (i, 0)),
                  pl.BlockSpec((NB, 1), lambda i: (i, 0)),
                  pl.BlockSpec((1, H), lambda i: (0, 0))],
        out_specs=[pl.BlockSpec((NB, H), lambda i: (i, 0)),
                   pl.BlockSpec((2, H), lambda i: (0, 0))],
        out_shape=[jax.ShapeDtypeStruct((NP, H), _f32),
                   jax.ShapeDtypeStruct((2, H), _f32)],
    )(agg, xwp, dinv, bias)


def _next_body(t_ref, res_ref, a_ref, c_ref, w_ref, dinv_ref, h_ref, xw_ref):
    h = jnp.maximum(t_ref[...] * a_ref[...] + c_ref[...], 0.0) + res_ref[...]
    h_ref[...] = h
    xw_ref[...] = jnp.dot(h, w_ref[...],
                          preferred_element_type=_f32) * dinv_ref[...]


def _tc_next(t, res, a, c, w, dinv):
    return pl.pallas_call(
        _next_body,
        grid=(GRID,),
        in_specs=[pl.BlockSpec((NB, H), lambda i: (i, 0)),
                  pl.BlockSpec((NB, H), lambda i: (i, 0)),
                  pl.BlockSpec((1, H), lambda i: (0, 0)),
                  pl.BlockSpec((1, H), lambda i: (0, 0)),
                  pl.BlockSpec((H, H), lambda i: (0, 0)),
                  pl.BlockSpec((NB, 1), lambda i: (i, 0))],
        out_specs=[pl.BlockSpec((NB, H), lambda i: (i, 0)),
                   pl.BlockSpec((NB, H), lambda i: (i, 0))],
        out_shape=[jax.ShapeDtypeStruct((NP, H), _f32),
                   jax.ShapeDtypeStruct((NP, H), _f32)],
    )(t, res, a, c, w, dinv)


def _ro_body(t_ref, res_ref, a_ref, c_ref, bi_ref, lw_ref, lb_ref,
             o_ref, acc_s, acc_c):
    i = pl.program_id(0)
    h = t_ref[...] * a_ref[...] + c_ref[...] + res_ref[...]
    rows = i * NB + lax.broadcasted_iota(_i32, (NB, 1), 0)
    m = rows < N
    oh = jnp.where((bi_ref[...] == lax.broadcasted_iota(_i32, (NB, G), 1)) & m,
                   1.0, 0.0)

    @pl.when(i == 0)
    def _():
        acc_s[...] = jnp.zeros_like(acc_s)
        acc_c[...] = jnp.zeros_like(acc_c)

    dn = (((0,), (0,)), ((), ()))
    acc_s[...] += lax.dot_general(oh, h, dn, preferred_element_type=_f32)
    acc_c[...] += lax.dot_general(oh, jnp.ones((NB, 1), _f32), dn,
                                  preferred_element_type=_f32)

    @pl.when(i == GRID - 1)
    def _():
        g = acc_s[...] / jnp.maximum(acc_c[...], 1.0)
        o_ref[...] = jax.nn.sigmoid(
            jnp.dot(g, lw_ref[...], preferred_element_type=_f32) + lb_ref[...])


def _tc_readout(t, res, a, c, bi, lw, lb):
    return pl.pallas_call(
        _ro_body,
        grid=(GRID,),
        in_specs=[pl.BlockSpec((NB, H), lambda i: (i, 0)),
                  pl.BlockSpec((NB, H), lambda i: (i, 0)),
                  pl.BlockSpec((1, H), lambda i: (0, 0)),
                  pl.BlockSpec((1, H), lambda i: (0, 0)),
                  pl.BlockSpec((NB, 1), lambda i: (i, 0)),
                  pl.BlockSpec((H, 1), lambda i: (0, 0)),
                  pl.BlockSpec((1, 1), lambda i: (0, 0))],
        out_specs=pl.BlockSpec((G, 1), lambda i: (0, 0)),
        out_shape=jax.ShapeDtypeStruct((G, 1), _f32),
        scratch_shapes=[pltpu.VMEM((G, H), _f32), pltpu.VMEM((G, 1), _f32)],
    )(t, res, a, c, bi, lw, lb)


def kernel(x, edge_index, batch_idx, emb, W, b, gamma, beta, lin_W, lin_b):
    xt = jnp.pad(x.astype(_i32).T, ((0, 0), (0, NP - N))).reshape(-1)
    embf = emb.reshape(F * V, H)
    src2d = jnp.pad(edge_index[0].astype(_i32), (0, EPA - E),
                    constant_values=N).reshape(EPA // 128, 128)
    dst2d = jnp.pad(edge_index[1].astype(_i32), (0, EPA - E),
                    constant_values=N).reshape(EPA // 128, 128)
    bi = jnp.pad(batch_idx.astype(_i32), (0, NP - N)).reshape(NP, 1)

    dstf = jnp.pad(edge_index[1].astype(_i32), (0, EP - E),
                   constant_values=N).reshape(EP, 1)
    h0 = _sc1(xt, embf)
    deg = _tc_deg(dstf).reshape(NP) + 1.0
    dinv = lax.rsqrt(deg).reshape(NP, 1)

    xwp = _tc_matmul(h0, W[0], dinv)
    hprev = h0
    for l in range(3):
        agg = _sc_agg(xwp, src2d, dst2d)
        t, sums = _tc_stats(agg, xwp, dinv, b[l].reshape(1, H))
        mu = sums[0] / N
        var = sums[1] / N - mu * mu
        aff_a = (gamma[l] * lax.rsqrt(var + 1e-5)).reshape(1, H)
        aff_c = (beta[l] - mu * gamma[l] * lax.rsqrt(var + 1e-5)).reshape(1, H)
        if l < 2:
            hprev, xwp = _tc_next(t, hprev, aff_a, aff_c, W[l + 1], dinv)
        else:
            out = _tc_readout(t, hprev, aff_a, aff_c, bi,
                              lin_W, lin_b.reshape(1, 1))
    return out
